# baseline (speedup 1.0000x reference)
import jax
import jax.numpy as jnp
from jax.experimental import pallas as pl
from jax.experimental.pallas import tpu as pltpu
from jax.experimental.pallas import tpu_sc as plsc

_ROWS = 2 * 8192
_COLS = 2048

_SC_ROWS = 8192
_TC_ROWS = _ROWS - _SC_ROWS
_TC_BLK = 512

_NC = 2
_NS = 16
_NW = _NC * _NS
_L = 16
_SC_BLK_R = 8
_SC_GRID = _SC_ROWS // _SC_BLK_R
_SC_BASE = _TC_ROWS // _SC_BLK_R


def _tc_kernel(x_ref, y_ref, o_ref, acc_ref):
    i = pl.program_id(0)

    @pl.when(i == 0)
    def _init():
        acc_ref[0] = 0.0
        acc_ref[1] = 0.0

    x = x_ref[...]
    y = y_ref[...]
    d = x - y
    acc_ref[0] += jnp.sum(d * d)
    acc_ref[1] += jnp.sum(y * y)

    @pl.when(i == pl.num_programs(0) - 1)
    def _fin():
        o_ref[0] = acc_ref[0]
        o_ref[1] = acc_ref[1]


def _tc_partials(xf, yf):
    return pl.pallas_call(
        _tc_kernel,
        grid=(_TC_ROWS // _TC_BLK,),
        in_specs=[
            pl.BlockSpec((_TC_BLK, _COLS), lambda i: (i, 0)),
            pl.BlockSpec((_TC_BLK, _COLS), lambda i: (i, 0)),
        ],
        out_specs=pl.BlockSpec(memory_space=pltpu.SMEM),
        out_shape=jax.ShapeDtypeStruct((2,), jnp.float32),
        scratch_shapes=[pltpu.SMEM((2,), jnp.float32)],
    )(xf, yf)


def _sc_body(acc_l, acc_n, x_vmem, y_vmem):
    zeros = jnp.zeros((_L,), jnp.float32)

    def step(c, carry):
        al0, al1, an0, an1 = carry
        for r in range(_SC_BLK_R):
            xv = x_vmem[r, pl.ds(c * _L, _L)]
            yv = y_vmem[r, pl.ds(c * _L, _L)]
            d = xv - yv
            if r % 2 == 0:
                al0 = al0 + d * d
                an0 = an0 + yv * yv
            else:
                al1 = al1 + d * d
                an1 = an1 + yv * yv
        return (al0, al1, an0, an1)

    al0, al1, an0, an1 = plsc.parallel_loop(
        0, _COLS // _L, unroll=4, carry=(zeros, zeros, zeros, zeros))(step)
    acc_l[...] += al0 + al1
    acc_n[...] += an0 + an1


def _sc_partials(xf, yf):
    mesh = plsc.VectorSubcoreMesh(core_axis_name="c", subcore_axis_name="s")

    @pl.kernel(
        out_type=jax.ShapeDtypeStruct((_NW, 2, _L), jnp.float32),
        mesh=mesh,
        scratch_types=[
            pltpu.VMEM((_L,), jnp.float32),
            pltpu.VMEM((_L,), jnp.float32),
        ],
    )
    def k(x_hbm, y_hbm, o_hbm, acc_l, acc_n):
        acc_l[...] = jnp.zeros((_L,), jnp.float32)
        acc_n[...] = jnp.zeros((_L,), jnp.float32)

        pltpu.emit_pipeline(
            lambda xv, yv: _sc_body(acc_l, acc_n, xv, yv),
            grid=(_SC_GRID,),
            in_specs=[
                pl.BlockSpec((_SC_BLK_R, _COLS), lambda i: (i + _SC_BASE, 0)),
                pl.BlockSpec((_SC_BLK_R, _COLS), lambda i: (i + _SC_BASE, 0)),
            ],
            out_specs=[],
            core_axis_name=("c", "s"),
            dimension_semantics=(pltpu.PARALLEL,),
        )(x_hbm, y_hbm)

        wid = jax.lax.axis_index("s") * _NC + jax.lax.axis_index("c")
        pltpu.sync_copy(acc_l, o_hbm.at[wid, 0])
        pltpu.sync_copy(acc_n, o_hbm.at[wid, 1])

    return k(xf, yf)


def kernel(x, y):
    xf = x.reshape(_ROWS, _COLS)
    yf = y.reshape(_ROWS, _COLS)
    sc_p = _sc_partials(xf, yf)
    tc_p = _tc_partials(xf, yf)
    loss_sum = tc_p[0] + jnp.sum(sc_p[:, 0, :])
    norm_sum = tc_p[1] + jnp.sum(sc_p[:, 1, :])
    return loss_sum / norm_sum


# speedup vs baseline: 1.0297x; 1.0297x over previous
import jax
import jax.numpy as jnp
from jax.experimental import pallas as pl
from jax.experimental.pallas import tpu as pltpu
from jax.experimental.pallas import tpu_sc as plsc

_ROWS = 2 * 8192
_COLS = 2048

_SC_ROWS = 4096
_TC_ROWS = _ROWS - _SC_ROWS
_TC_BLK = 1024

_NC = 2
_NS = 16
_NW = _NC * _NS
_L = 16
_SC_BLK_R = 8
_SC_GRID = _SC_ROWS // _SC_BLK_R
_SC_BASE = _TC_ROWS // _SC_BLK_R


def _tc_kernel(x_ref, y_ref, o_ref, acc_ref):
    i = pl.program_id(0)

    @pl.when(i == 0)
    def _init():
        acc_ref[0] = 0.0
        acc_ref[1] = 0.0

    x = x_ref[...]
    y = y_ref[...]
    d = x - y
    acc_ref[0] += jnp.sum(d * d)
    acc_ref[1] += jnp.sum(y * y)

    @pl.when(i == pl.num_programs(0) - 1)
    def _fin():
        o_ref[0] = acc_ref[0]
        o_ref[1] = acc_ref[1]


def _tc_partials(xf, yf):
    return pl.pallas_call(
        _tc_kernel,
        grid=(_TC_ROWS // _TC_BLK,),
        in_specs=[
            pl.BlockSpec((_TC_BLK, _COLS), lambda i: (i, 0)),
            pl.BlockSpec((_TC_BLK, _COLS), lambda i: (i, 0)),
        ],
        out_specs=pl.BlockSpec(memory_space=pltpu.SMEM),
        out_shape=jax.ShapeDtypeStruct((2,), jnp.float32),
        scratch_shapes=[pltpu.SMEM((2,), jnp.float32)],
    )(xf, yf)


def _sc_body(acc_l, acc_n, x_vmem, y_vmem):
    zeros = jnp.zeros((_L,), jnp.float32)

    def step(c, carry):
        al0, al1, an0, an1 = carry
        for r in range(_SC_BLK_R):
            xv = x_vmem[r, pl.ds(c * _L, _L)]
            yv = y_vmem[r, pl.ds(c * _L, _L)]
            d = xv - yv
            if r % 2 == 0:
                al0 = al0 + d * d
                an0 = an0 + yv * yv
            else:
                al1 = al1 + d * d
                an1 = an1 + yv * yv
        return (al0, al1, an0, an1)

    al0, al1, an0, an1 = plsc.parallel_loop(
        0, _COLS // _L, unroll=4, carry=(zeros, zeros, zeros, zeros))(step)
    acc_l[...] += al0 + al1
    acc_n[...] += an0 + an1


def _sc_partials(xf, yf):
    mesh = plsc.VectorSubcoreMesh(core_axis_name="c", subcore_axis_name="s")

    @pl.kernel(
        out_type=jax.ShapeDtypeStruct((_NW, 2, _L), jnp.float32),
        mesh=mesh,
        scratch_types=[
            pltpu.VMEM((_L,), jnp.float32),
            pltpu.VMEM((_L,), jnp.float32),
        ],
    )
    def k(x_hbm, y_hbm, o_hbm, acc_l, acc_n):
        acc_l[...] = jnp.zeros((_L,), jnp.float32)
        acc_n[...] = jnp.zeros((_L,), jnp.float32)

        pltpu.emit_pipeline(
            lambda xv, yv: _sc_body(acc_l, acc_n, xv, yv),
            grid=(_SC_GRID,),
            in_specs=[
                pl.BlockSpec((_SC_BLK_R, _COLS), lambda i: (i + _SC_BASE, 0)),
                pl.BlockSpec((_SC_BLK_R, _COLS), lambda i: (i + _SC_BASE, 0)),
            ],
            out_specs=[],
            core_axis_name=("c", "s"),
            dimension_semantics=(pltpu.PARALLEL,),
        )(x_hbm, y_hbm)

        wid = jax.lax.axis_index("s") * _NC + jax.lax.axis_index("c")
        pltpu.sync_copy(acc_l, o_hbm.at[wid, 0])
        pltpu.sync_copy(acc_n, o_hbm.at[wid, 1])

    return k(xf, yf)


def kernel(x, y):
    xf = x.reshape(_ROWS, _COLS)
    yf = y.reshape(_ROWS, _COLS)
    sc_p = _sc_partials(xf, yf)
    tc_p = _tc_partials(xf, yf)
    loss_sum = tc_p[0] + jnp.sum(sc_p[:, 0, :])
    norm_sum = tc_p[1] + jnp.sum(sc_p[:, 1, :])
    return loss_sum / norm_sum


# speedup vs baseline: 1.0381x; 1.0082x over previous
import jax
import jax.numpy as jnp
from jax.experimental import pallas as pl
from jax.experimental.pallas import tpu as pltpu
from jax.experimental.pallas import tpu_sc as plsc

_ROWS = 2 * 8192
_COLS = 2048

_SC_ROWS = 512
_TC_ROWS = _ROWS - _SC_ROWS
_TC_BLK = 512

_NC = 2
_NS = 16
_NW = _NC * _NS
_L = 16
_SC_BLK_R = 8
_SC_GRID = _SC_ROWS // _SC_BLK_R
_SC_BASE = _TC_ROWS // _SC_BLK_R


def _tc_kernel(x_ref, y_ref, o_ref, acc_ref):
    i = pl.program_id(0)

    @pl.when(i == 0)
    def _init():
        acc_ref[0] = 0.0
        acc_ref[1] = 0.0

    x = x_ref[...]
    y = y_ref[...]
    d = x - y
    acc_ref[0] += jnp.sum(d * d)
    acc_ref[1] += jnp.sum(y * y)

    @pl.when(i == pl.num_programs(0) - 1)
    def _fin():
        o_ref[0] = acc_ref[0]
        o_ref[1] = acc_ref[1]


def _tc_partials(xf, yf):
    return pl.pallas_call(
        _tc_kernel,
        grid=(_TC_ROWS // _TC_BLK,),
        in_specs=[
            pl.BlockSpec((_TC_BLK, _COLS), lambda i: (i, 0)),
            pl.BlockSpec((_TC_BLK, _COLS), lambda i: (i, 0)),
        ],
        out_specs=pl.BlockSpec(memory_space=pltpu.SMEM),
        out_shape=jax.ShapeDtypeStruct((2,), jnp.float32),
        scratch_shapes=[pltpu.SMEM((2,), jnp.float32)],
    )(xf, yf)


def _sc_body(acc_l, acc_n, x_vmem, y_vmem):
    zeros = jnp.zeros((_L,), jnp.float32)

    def step(c, carry):
        al0, al1, an0, an1 = carry
        for r in range(_SC_BLK_R):
            xv = x_vmem[r, pl.ds(c * _L, _L)]
            yv = y_vmem[r, pl.ds(c * _L, _L)]
            d = xv - yv
            if r % 2 == 0:
                al0 = al0 + d * d
                an0 = an0 + yv * yv
            else:
                al1 = al1 + d * d
                an1 = an1 + yv * yv
        return (al0, al1, an0, an1)

    al0, al1, an0, an1 = plsc.parallel_loop(
        0, _COLS // _L, unroll=4, carry=(zeros, zeros, zeros, zeros))(step)
    acc_l[...] += al0 + al1
    acc_n[...] += an0 + an1


def _sc_partials(xf, yf):
    mesh = plsc.VectorSubcoreMesh(core_axis_name="c", subcore_axis_name="s")

    @pl.kernel(
        out_type=jax.ShapeDtypeStruct((_NW, 2, _L), jnp.float32),
        mesh=mesh,
        scratch_types=[
            pltpu.VMEM((_L,), jnp.float32),
            pltpu.VMEM((_L,), jnp.float32),
        ],
    )
    def k(x_hbm, y_hbm, o_hbm, acc_l, acc_n):
        acc_l[...] = jnp.zeros((_L,), jnp.float32)
        acc_n[...] = jnp.zeros((_L,), jnp.float32)

        pltpu.emit_pipeline(
            lambda xv, yv: _sc_body(acc_l, acc_n, xv, yv),
            grid=(_SC_GRID,),
            in_specs=[
                pl.BlockSpec((_SC_BLK_R, _COLS), lambda i: (i + _SC_BASE, 0)),
                pl.BlockSpec((_SC_BLK_R, _COLS), lambda i: (i + _SC_BASE, 0)),
            ],
            out_specs=[],
            core_axis_name=("c", "s"),
            dimension_semantics=(pltpu.PARALLEL,),
        )(x_hbm, y_hbm)

        wid = jax.lax.axis_index("s") * _NC + jax.lax.axis_index("c")
        pltpu.sync_copy(acc_l, o_hbm.at[wid, 0])
        pltpu.sync_copy(acc_n, o_hbm.at[wid, 1])

    return k(xf, yf)


def kernel(x, y):
    xf = x.reshape(_ROWS, _COLS)
    yf = y.reshape(_ROWS, _COLS)
    sc_p = _sc_partials(xf, yf)
    tc_p = _tc_partials(xf, yf)
    loss_sum = tc_p[0] + jnp.sum(sc_p[:, 0, :])
    norm_sum = tc_p[1] + jnp.sum(sc_p[:, 1, :])
    return loss_sum / norm_sum


# speedup vs baseline: 1.3440x; 1.2947x over previous
import jax
import jax.numpy as jnp
from jax.experimental import pallas as pl
from jax.experimental.pallas import tpu as pltpu

_ROWS = 2 * 8192
_COLS = 2048
_BLK = 512
_NBUF = 4
_NSTEP = _ROWS // _BLK


def _start(x_hbm, y_hbm, xb, yb, sems, j, s):
    pltpu.make_async_copy(
        x_hbm.at[pl.ds(j * _BLK, _BLK)], xb.at[s], sems.at[s, 0]).start()
    pltpu.make_async_copy(
        y_hbm.at[pl.ds(j * _BLK, _BLK)], yb.at[s], sems.at[s, 1]).start()


def _wait(x_hbm, y_hbm, xb, yb, sems, s):
    pltpu.make_async_copy(
        x_hbm.at[pl.ds(0, _BLK)], xb.at[s], sems.at[s, 0]).wait()
    pltpu.make_async_copy(
        y_hbm.at[pl.ds(0, _BLK)], yb.at[s], sems.at[s, 1]).wait()


def _reduce_kernel(x_hbm, y_hbm, o_ref, xb, yb, sems, acc_ref):
    i = pl.program_id(0)

    @pl.when(i == 0)
    def _init():
        acc_ref[0] = 0.0
        acc_ref[1] = 0.0
        for k in range(_NBUF - 1):
            _start(x_hbm, y_hbm, xb, yb, sems, k, k)

    j = i + _NBUF - 1

    @pl.when(j < _NSTEP)
    def _prefetch():
        _start(x_hbm, y_hbm, xb, yb, sems, j, j % _NBUF)

    s = i % _NBUF
    _wait(x_hbm, y_hbm, xb, yb, sems, s)
    x = xb[s]
    y = yb[s]
    d = x - y
    acc_ref[0] += jnp.sum(d * d)
    acc_ref[1] += jnp.sum(y * y)

    @pl.when(i == _NSTEP - 1)
    def _fin():
        o_ref[0] = acc_ref[0] / acc_ref[1]


def kernel(x, y):
    xf = x.reshape(_ROWS, _COLS)
    yf = y.reshape(_ROWS, _COLS)
    out = pl.pallas_call(
        _reduce_kernel,
        grid=(_NSTEP,),
        in_specs=[
            pl.BlockSpec(memory_space=pl.ANY),
            pl.BlockSpec(memory_space=pl.ANY),
        ],
        out_specs=pl.BlockSpec(memory_space=pltpu.SMEM),
        out_shape=jax.ShapeDtypeStruct((1,), jnp.float32),
        scratch_shapes=[
            pltpu.VMEM((_NBUF, _BLK, _COLS), jnp.float32),
            pltpu.VMEM((_NBUF, _BLK, _COLS), jnp.float32),
            pltpu.SemaphoreType.DMA((_NBUF, 2)),
            pltpu.SMEM((2,), jnp.float32),
        ],
        compiler_params=pltpu.CompilerParams(
            dimension_semantics=("arbitrary",)),
    )(xf, yf)
    return out[0]


# speedup vs baseline: 1.3462x; 1.0016x over previous
import jax
import jax.numpy as jnp
from jax.experimental import pallas as pl
from jax.experimental.pallas import tpu as pltpu

_ROWS = 2 * 8192
_COLS = 2048
_BLK = 256
_NBUF = 8
_NSTEP = _ROWS // _BLK


def _start(x_hbm, y_hbm, xb, yb, sems, j, s):
    pltpu.make_async_copy(
        x_hbm.at[pl.ds(j * _BLK, _BLK)], xb.at[s], sems.at[s, 0]).start()
    pltpu.make_async_copy(
        y_hbm.at[pl.ds(j * _BLK, _BLK)], yb.at[s], sems.at[s, 1]).start()


def _wait(x_hbm, y_hbm, xb, yb, sems, s):
    pltpu.make_async_copy(
        x_hbm.at[pl.ds(0, _BLK)], xb.at[s], sems.at[s, 0]).wait()
    pltpu.make_async_copy(
        y_hbm.at[pl.ds(0, _BLK)], yb.at[s], sems.at[s, 1]).wait()


def _reduce_kernel(x_hbm, y_hbm, o_ref, xb, yb, sems, acc_ref):
    i = pl.program_id(0)

    @pl.when(i == 0)
    def _init():
        acc_ref[0] = 0.0
        acc_ref[1] = 0.0
        for k in range(_NBUF - 1):
            _start(x_hbm, y_hbm, xb, yb, sems, k, k)

    j = i + _NBUF - 1

    @pl.when(j < _NSTEP)
    def _prefetch():
        _start(x_hbm, y_hbm, xb, yb, sems, j, j % _NBUF)

    s = i % _NBUF
    _wait(x_hbm, y_hbm, xb, yb, sems, s)
    x = xb[s]
    y = yb[s]
    d = x - y
    acc_ref[0] += jnp.sum(d * d)
    acc_ref[1] += jnp.sum(y * y)

    @pl.when(i == _NSTEP - 1)
    def _fin():
        o_ref[0] = acc_ref[0] / acc_ref[1]


def kernel(x, y):
    xf = x.reshape(_ROWS, _COLS)
    yf = y.reshape(_ROWS, _COLS)
    out = pl.pallas_call(
        _reduce_kernel,
        grid=(_NSTEP,),
        in_specs=[
            pl.BlockSpec(memory_space=pl.ANY),
            pl.BlockSpec(memory_space=pl.ANY),
        ],
        out_specs=pl.BlockSpec(memory_space=pltpu.SMEM),
        out_shape=jax.ShapeDtypeStruct((1,), jnp.float32),
        scratch_shapes=[
            pltpu.VMEM((_NBUF, _BLK, _COLS), jnp.float32),
            pltpu.VMEM((_NBUF, _BLK, _COLS), jnp.float32),
            pltpu.SemaphoreType.DMA((_NBUF, 2)),
            pltpu.SMEM((2,), jnp.float32),
        ],
        compiler_params=pltpu.CompilerParams(
            dimension_semantics=("arbitrary",)),
    )(xf, yf)
    return out[0]
